# baseline (speedup 1.0000x reference)
import functools
import math

import jax
import jax.numpy as jnp
from jax.experimental import pallas as pl
from jax.experimental.pallas import tpu as pltpu

N_HEADS = 16
HEAD_DIM = 128
LOG2E = 1.4426950408889634


def _matmul_bias_kernel(x_ref, w_ref, b_ref, o_ref):
    acc = jax.lax.dot_general(
        x_ref[...],
        w_ref[...].astype(jnp.bfloat16),
        (((1,), (1,)), ((), ())),
        preferred_element_type=jnp.float32,
    ) + b_ref[...]
    o_ref[...] = acc.astype(o_ref.dtype)


def _matmul_bias(x, w, b, bn, out_dtype):
    t, k = x.shape
    n = w.shape[0]
    grid = (n // bn,)
    return pl.pallas_call(
        _matmul_bias_kernel,
        grid=grid,
        in_specs=[
            pl.BlockSpec((t, k), lambda j: (0, 0)),
            pl.BlockSpec((bn, k), lambda j: (j, 0)),
            pl.BlockSpec((1, bn), lambda j: (0, j)),
        ],
        out_specs=pl.BlockSpec((t, bn), lambda j: (0, j)),
        out_shape=jax.ShapeDtypeStruct((t, n), out_dtype),
        compiler_params=pltpu.CompilerParams(
            dimension_semantics=("parallel",),
        ),
    )(x, w, b.reshape(1, n))


def _mega_kernel(x_ref, w_ref, b_ref, o_ref, qkv_ref, vaug_ref,
                 *, t, bn, nmm, bq, bk, scale):
    j = pl.program_id(0)
    hs = HEAD_DIM
    nq = t // bq
    hcols = bn // hs

    @pl.when(j < nmm)
    def _():
        acc = jax.lax.dot_general(
            x_ref[...],
            w_ref[...].astype(jnp.bfloat16),
            (((1,), (1,)), ((), ())),
            preferred_element_type=jnp.float32,
        ) + b_ref[...]
        ab = acc.astype(jnp.bfloat16)
        for m in range(hcols):
            qkv_ref[hcols * j + m] = ab[:, m * hs:(m + 1) * hs]

    @pl.when(j >= nmm)
    def _():
        h = j - nmm
        qb = qkv_ref[h]
        kb = qkv_ref[N_HEADS + h]
        vb = qkv_ref[2 * N_HEADS + h]

        ones_h = jnp.ones((hs, 128), jnp.bfloat16)
        qn = jax.lax.dot_general(
            qb * qb, ones_h, (((1,), (0,)), ((), ())),
            preferred_element_type=jnp.float32,
        )
        kn = jax.lax.dot_general(
            kb * kb, ones_h, (((1,), (0,)), ((), ())),
            preferred_element_type=jnp.float32,
        )
        m_r = jnp.sqrt(jnp.max(qn)) * jnp.sqrt(jnp.max(kn)) * (scale * 1.05)
        c1 = jnp.float32(scale * LOG2E)
        c2 = m_r * jnp.float32(LOG2E)

        vaug_ref[:, :hs] = vb
        vaug_ref[:, hs:] = jnp.ones((t, hs), jnp.bfloat16)

        rows = jax.lax.broadcasted_iota(jnp.int32, (bq, bk), 0)
        cols = jax.lax.broadcasted_iota(jnp.int32, (bq, bk), 1)
        diag_mask = rows >= cols

        for ib in range(nq):
            q = qb[ib * bq:(ib + 1) * bq, :]
            acc = None
            for jj in range(ib + 1):
                kc = kb[jj * bk:(jj + 1) * bk, :]
                s = jax.lax.dot_general(
                    q, kc, (((1,), (1,)), ((), ())),
                    preferred_element_type=jnp.float32,
                )
                p = jnp.exp2(s * c1 - c2)
                if jj == ib:
                    p = jnp.where(diag_mask, p, 0.0)
                vc = vaug_ref[jj * bk:(jj + 1) * bk, :]
                pv = jax.lax.dot_general(
                    p.astype(jnp.bfloat16), vc, (((1,), (0,)), ((), ())),
                    preferred_element_type=jnp.float32,
                )
                acc = pv if acc is None else acc + pv
            o_ref[ib * bq:(ib + 1) * bq, :] = (
                acc[:, :hs] / acc[:, hs:]).astype(o_ref.dtype)


def _qkv_flash(x2, W_attn, b_attn, t, c, bn, bq, bk):
    hs = HEAD_DIM
    nmm = (3 * c) // bn
    scale = 1.0 / math.sqrt(hs)
    kern = functools.partial(
        _mega_kernel, t=t, bn=bn, nmm=nmm, bq=bq, bk=bk, scale=scale)
    nsteps = nmm + N_HEADS
    return pl.pallas_call(
        kern,
        grid=(nsteps,),
        in_specs=[
            pl.BlockSpec((t, c), lambda j: (0, 0)),
            pl.BlockSpec((bn, c), lambda j: (jnp.minimum(j, nmm - 1), 0)),
            pl.BlockSpec((1, bn), lambda j: (0, jnp.minimum(j, nmm - 1))),
        ],
        out_specs=pl.BlockSpec(
            (t, hs), lambda j: (0, jnp.maximum(j - nmm, 0))),
        out_shape=jax.ShapeDtypeStruct((t, c), jnp.bfloat16),
        scratch_shapes=[
            pltpu.VMEM((3 * N_HEADS, t, hs), jnp.bfloat16),
            pltpu.VMEM((t, 2 * hs), jnp.bfloat16),
        ],
        compiler_params=pltpu.CompilerParams(
            dimension_semantics=("arbitrary",),
        ),
    )(x2, W_attn, b_attn.reshape(1, 3 * c))


@jax.jit
def _attention_impl(x, W_attn, b_attn, W_proj, b_proj):
    b, t, c = x.shape
    x2 = x.reshape(t, c).astype(jnp.bfloat16)
    y = _qkv_flash(x2, W_attn, b_attn, t, c, bn=512, bq=512, bk=512)
    out = _matmul_bias(y, W_proj, b_proj, bn=512, out_dtype=jnp.float32)
    return out.reshape(b, t, c)


def kernel(x, W_attn, b_attn, W_proj, b_proj):
    return _attention_impl(x, W_attn, b_attn, W_proj, b_proj)
